# baseline (speedup 1.0000x reference)
import functools

import jax
from jax import lax
import jax.numpy as jnp
from jax.experimental import pallas as pl
from jax.experimental.pallas import tpu as pltpu
from jax.experimental.pallas import tpu_sc as plsc

_NPOINT = 1024
_RADIUS = 0.2
_NSAMPLE = 32
_B, _N, _CFEAT = 8, 4096, 64
_MLP_DIMS = [128, 128, 256]
_IN_CH = 3 + _CFEAT


def _fps_body(xyzt_ref, cx_ref, cy_ref, cz_ref, dist_ref):
    x = xyzt_ref[0]
    y = xyzt_ref[1]
    z = xyzt_ref[2]
    iota_n = jax.lax.broadcasted_iota(jnp.int32, (_B, _N), 1)
    iota_s = jax.lax.broadcasted_iota(jnp.int32, (_B, _NPOINT), 1)
    dist_ref[...] = jnp.full((_B, _N), 1e10, jnp.float32)

    def body(i, carry):
        far, cxa, cya, cza = carry
        onehot = (iota_n == far).astype(jnp.float32)
        cx = jnp.sum(x * onehot, axis=1, keepdims=True)
        cy = jnp.sum(y * onehot, axis=1, keepdims=True)
        cz = jnp.sum(z * onehot, axis=1, keepdims=True)
        sel = iota_s == i
        cxa = jnp.where(sel, cx, cxa)
        cya = jnp.where(sel, cy, cya)
        cza = jnp.where(sel, cz, cza)
        dx = x - cx
        dy = y - cy
        dz = z - cz
        d = (dx * dx + dy * dy) + dz * dz
        dmin = jnp.minimum(dist_ref[...], d)
        dist_ref[...] = dmin
        m = jnp.max(dmin, axis=1, keepdims=True)
        far = jnp.min(jnp.where(dmin == m, iota_n, _N), axis=1, keepdims=True)
        return far, cxa, cya, cza

    far0 = jnp.zeros((_B, 1), jnp.int32)
    zeros_s = jnp.zeros((_B, _NPOINT), jnp.float32)
    _, cxa, cya, cza = jax.lax.fori_loop(
        0, _NPOINT, body, (far0, zeros_s, zeros_s, zeros_s))
    cx_ref[...] = cxa
    cy_ref[...] = cya
    cz_ref[...] = cza


def _run_fps(xyzt, interpret=False):
    cx, cy, cz = pl.pallas_call(
        _fps_body,
        out_shape=[jax.ShapeDtypeStruct((_B, _NPOINT), jnp.float32)] * 3,
        scratch_shapes=[pltpu.VMEM((_B, _N), jnp.float32)],
        interpret=interpret,
    )(xyzt)
    return cx, cy, cz


_SBLK = 256


def _mask_body(qp_ref, pp_ref, lo_ref, hi_ref, out_ref):
    qp = qp_ref[0]
    pp = pp_ref[0]
    qx = qp[:, 0:1]
    qy = qp[:, 1:2]
    qz = qp[:, 2:3]
    qn2 = (qx * qx + qy * qy) + qz * qz
    px = pp[0:1, :]
    py = pp[1:2, :]
    pz = pp[2:3, :]
    pn2 = (px * px + py * py) + pz * pz
    dot = jnp.dot(qp, pp, preferred_element_type=jnp.float32)
    sqr = (qn2 + pn2) - 2.0 * dot
    bit = jnp.where(sqr > _R2, 0.0, 1.0).astype(jnp.bfloat16)
    lo = jnp.dot(bit, lo_ref[...], preferred_element_type=jnp.float32)
    hi = jnp.dot(bit, hi_ref[...], preferred_element_type=jnp.float32)
    out_ref[0] = lo.astype(jnp.int32) | (hi.astype(jnp.int32) << 16)


def _pack_bases():
    n = jnp.arange(_N, dtype=jnp.int32)
    t = jnp.arange(_N // 32, dtype=jnp.int32)
    block = (n[:, None] // 32) == t[None, :]
    jmod = (n % 32)[:, None]
    pw = (1 << (jmod % 16)).astype(jnp.float32)
    lo = jnp.where(block & (jmod < 16), pw, 0.0)
    hi = jnp.where(block & (jmod >= 16), pw, 0.0)
    return lo.astype(jnp.bfloat16), hi.astype(jnp.bfloat16)


def _run_mask(q_pad, p_pad):
    lo, hi = _pack_bases()
    return pl.pallas_call(
        _mask_body,
        grid=(_B, _NPOINT // _SBLK),
        in_specs=[
            pl.BlockSpec((1, _SBLK, 8), lambda b, s: (b, s, 0)),
            pl.BlockSpec((1, 8, _N), lambda b, s: (b, 0, 0)),
            pl.BlockSpec((_N, _N // 32), lambda b, s: (0, 0)),
            pl.BlockSpec((_N, _N // 32), lambda b, s: (0, 0)),
        ],
        out_specs=pl.BlockSpec((1, _SBLK, _N // 32), lambda b, s: (b, s, 0)),
        out_shape=jax.ShapeDtypeStruct((_B, _NPOINT, _N // 32), jnp.int32),
    )(q_pad, p_pad, lo, hi)


_NC, _NS, _L = 2, 16, 16
_NW = _NC * _NS
_QPW = _NPOINT * _B // _NW
_RPW = _QPW * _NSAMPLE
_R2 = _RADIUS * _RADIUS
_GCH = 64
_NBUF = 4


def _bq_body(words_hbm, px_hbm, py_hbm, pz_hbm, qx_hbm, qy_hbm, qz_hbm,
             pts_hbm, x1_hbm, x0_hbm,
             words_v, px_v, py_v, pz_v, qx_v, qy_v, qz_v, nbr_v, gidx_v, x0_v,
             bufs_v, gsems, wsems):
    wid = lax.axis_index("s") * _NC + lax.axis_index("c")
    b = wid // 4
    q0 = (wid % 4) * _QPW
    iota = lax.iota(jnp.int32, _L)
    nwords = _N // 32

    pltpu.sync_copy(words_hbm.at[b, pl.ds(q0, _QPW)], words_v)
    pltpu.sync_copy(px_hbm.at[b], px_v)
    pltpu.sync_copy(py_hbm.at[b], py_v)
    pltpu.sync_copy(pz_hbm.at[b], pz_v)
    pltpu.sync_copy(qx_hbm.at[b, pl.ds(q0, _QPW)], qx_v)
    pltpu.sync_copy(qy_hbm.at[b, pl.ds(q0, _QPW)], qy_v)
    pltpu.sync_copy(qz_hbm.at[b, pl.ds(q0, _QPW)], qz_v)

    for qv in range(_QPW // _L):
        qi = qv * _L + iota
        rowbase = qi * _NSAMPLE

        def scan_cond(carry):
            t, cnt = carry
            return (t < nwords) & (jnp.min(cnt) < _NSAMPLE)

        def scan_body(carry):
            t, cnt = carry
            tt = jnp.full((_L,), t, jnp.int32)
            w = plsc.load_gather(words_v, [qi, tt])
            nbase = t * 32
            for j in range(32):
                m = ((lax.shift_right_logical(w, j) & 1) == 1) \
                    & (cnt < _NSAMPLE)
                nn = jnp.full((_L,), nbase + j, jnp.int32)
                plsc.store_scatter(nbr_v, [rowbase + cnt], nn, mask=m)
                cnt = cnt + jnp.where(m, 1, 0)
            return t + 1, cnt

        _, cnt = lax.while_loop(scan_cond, scan_body,
                                (jnp.int32(0), jnp.zeros((_L,), jnp.int32)))
        first = plsc.load_gather(nbr_v, [rowbase])
        for k in range(1, _NSAMPLE):
            plsc.store_scatter(nbr_v, [rowbase + k], first,
                               mask=cnt <= k)

    def cbody(i, _):
        ii = i * _L + iota
        loc = plsc.load_gather(nbr_v, [ii])
        plsc.store_scatter(gidx_v, [ii], loc + b * _N)
        s_loc = lax.shift_right_logical(ii, 5)
        pxn = plsc.load_gather(px_v, [loc])
        pyn = plsc.load_gather(py_v, [loc])
        pzn = plsc.load_gather(pz_v, [loc])
        qxn = plsc.load_gather(qx_v, [s_loc])
        qyn = plsc.load_gather(qy_v, [s_loc])
        qzn = plsc.load_gather(qz_v, [s_loc])
        zero = jnp.zeros((_L,), jnp.float32)
        plsc.store_scatter(x0_v, [ii * 4], pxn - qxn)
        plsc.store_scatter(x0_v, [ii * 4 + 1], pyn - qyn)
        plsc.store_scatter(x0_v, [ii * 4 + 2], pzn - qzn)
        plsc.store_scatter(x0_v, [ii * 4 + 3], zero)
        return 0

    lax.fori_loop(0, _RPW // _L, cbody, 0)
    pltpu.sync_copy(x0_v, x0_hbm.at[pl.ds(wid * _RPW * 4, _RPW * 4)])

    nch = _RPW // _GCH
    gcp = [None] * _NBUF
    wcp = [None] * _NBUF
    for c in range(nch + 2):
        if c < nch:
            if c >= _NBUF:
                wcp[c % _NBUF].wait()
            gcp[c % _NBUF] = pltpu.async_copy(
                pts_hbm.at[gidx_v.at[pl.ds(c * _GCH, _GCH)]],
                bufs_v.at[c % _NBUF], gsems[c % _NBUF])
        if c >= 2:
            p = (c - 2) % _NBUF
            gcp[p].wait()
            wcp[p] = pltpu.async_copy(
                bufs_v.at[p],
                x1_hbm.at[pl.ds(wid * _RPW + (c - 2) * _GCH, _GCH)],
                wsems[p])
    for c in range(nch - _NBUF, nch):
        wcp[c % _NBUF].wait()


def _run_ball_group(words, xyzt, newx, newy, newz, points):
    pts_flat = points.reshape(_B * _N, _CFEAT)
    mesh = plsc.VectorSubcoreMesh(core_axis_name="c", subcore_axis_name="s",
                                  num_cores=_NC, num_subcores=_NS)
    x1, x0 = pl.kernel(
        _bq_body,
        compiler_params=pltpu.CompilerParams(needs_layout_passes=False,
                                             use_tc_tiling_on_sc=False),
        out_type=[
            jax.ShapeDtypeStruct((_B * _NPOINT * _NSAMPLE, _CFEAT),
                                 jnp.float32),
            jax.ShapeDtypeStruct((_B * _NPOINT * _NSAMPLE * 4,), jnp.float32),
        ],
        mesh=mesh,
        scratch_types=[
            pltpu.VMEM((_QPW, _N // 32), jnp.int32),
            pltpu.VMEM((_N,), jnp.float32),
            pltpu.VMEM((_N,), jnp.float32),
            pltpu.VMEM((_N,), jnp.float32),
            pltpu.VMEM((_QPW,), jnp.float32),
            pltpu.VMEM((_QPW,), jnp.float32),
            pltpu.VMEM((_QPW,), jnp.float32),
            pltpu.VMEM((_RPW,), jnp.int32),
            pltpu.VMEM((_RPW,), jnp.int32),
            pltpu.VMEM((_RPW * 4,), jnp.float32),
            pltpu.VMEM((_NBUF, _GCH, _CFEAT), jnp.float32),
            [pltpu.SemaphoreType.DMA] * _NBUF,
            [pltpu.SemaphoreType.DMA] * _NBUF,
        ],
    )(words, xyzt[0], xyzt[1], xyzt[2], newx, newy, newz, pts_flat)
    return x1, x0.reshape(_B * _NPOINT * _NSAMPLE, 4)


_RW = _B * _NPOINT * _NSAMPLE
_RBLK = 4096
_NRB = _RW // _RBLK


def _stats_update(st_ref, y):
    s = jnp.sum(y, axis=0, keepdims=True)
    s2 = jnp.sum(y * y, axis=0, keepdims=True)
    st = jnp.concatenate(
        [s, s2, jnp.zeros((6, y.shape[1]), jnp.float32)], axis=0)

    @pl.when(pl.program_id(0) == 0)
    def _():
        st_ref[...] = jnp.zeros_like(st_ref)

    st_ref[...] += st


def _norm_relu(y, st_ref, g_ref, be_ref):
    n = jnp.float32(_RW)
    mean = st_ref[0:1] / n
    var = st_ref[1:2] / n - mean * mean
    scale = g_ref[...] / jnp.sqrt(var + 1e-5)
    shift = be_ref[...] - mean * scale
    return jnp.maximum(y * scale + shift, 0.0)


def _mlp1_body(x0_ref, x1_ref, w0a_ref, w0b_ref, b0_ref, y_ref, st_ref):
    y = jnp.dot(x1_ref[...], w0b_ref[...],
                preferred_element_type=jnp.float32)
    y = y + jnp.dot(x0_ref[...], w0a_ref[...],
                    preferred_element_type=jnp.float32)
    y = y + b0_ref[...]
    y_ref[...] = y
    _stats_update(st_ref, y)


def _mlp_mid_body(y_ref, st_ref, g_ref, be_ref, w_ref, b_ref,
                  out_ref, st2_ref):
    x = _norm_relu(y_ref[...], st_ref, g_ref, be_ref)
    y = jnp.dot(x, w_ref[...], preferred_element_type=jnp.float32)
    y = y + b_ref[...]
    out_ref[...] = y
    _stats_update(st2_ref, y)


def _mlp3_body(y_ref, st_ref, g_ref, be_ref, w_ref, b_ref, st2_ref):
    x = _norm_relu(y_ref[...], st_ref, g_ref, be_ref)
    y = jnp.dot(x, w_ref[...], preferred_element_type=jnp.float32)
    y = y + b_ref[...]
    _stats_update(st2_ref, y)


def _mlp_tail_body(y_ref, st_ref, g_ref, be_ref, w_ref, b_ref,
                   st2_ref, g2_ref, be2_ref, out_ref):
    x = _norm_relu(y_ref[...], st_ref, g_ref, be_ref)
    y = jnp.dot(x, w_ref[...], preferred_element_type=jnp.float32)
    y = y + b_ref[...]
    x2 = _norm_relu(y, st2_ref, g2_ref, be2_ref)
    xg = x2.reshape(_RBLK // _NSAMPLE, _NSAMPLE, x2.shape[-1])
    out_ref[...] = jnp.max(xg, axis=1)


def _full(shape):
    return pl.BlockSpec(shape, lambda i: tuple(0 for _ in shape))


def _run_mlp(x0, x1, W0, b0, g0, be0, W1, b1, g1, be1, W2, b2, g2, be2):
    w0a = jnp.concatenate([W0[:3], jnp.zeros((1, 128), jnp.float32)], axis=0)
    w0b = W0[3:]
    y0, st0 = pl.pallas_call(
        _mlp1_body,
        grid=(_NRB,),
        in_specs=[pl.BlockSpec((_RBLK, 4), lambda i: (i, 0)),
                  pl.BlockSpec((_RBLK, _CFEAT), lambda i: (i, 0)),
                  _full((4, 128)), _full((_CFEAT, 128)), _full((1, 128))],
        out_specs=[pl.BlockSpec((_RBLK, 128), lambda i: (i, 0)),
                   _full((8, 128))],
        out_shape=[jax.ShapeDtypeStruct((_RW, 128), jnp.float32),
                   jax.ShapeDtypeStruct((8, 128), jnp.float32)],
    )(x0, x1, w0a, w0b, b0.reshape(1, 128))

    def mid(y, st, g, be, w, b, dout):
        din = y.shape[-1]
        return pl.pallas_call(
            _mlp_mid_body,
            grid=(_NRB,),
            in_specs=[pl.BlockSpec((_RBLK, din), lambda i: (i, 0)),
                      _full((8, din)), _full((1, din)), _full((1, din)),
                      _full((din, dout)), _full((1, dout))],
            out_specs=[pl.BlockSpec((_RBLK, dout), lambda i: (i, 0)),
                       _full((8, dout))],
            out_shape=[jax.ShapeDtypeStruct((_RW, dout), jnp.float32),
                       jax.ShapeDtypeStruct((8, dout), jnp.float32)],
        )(y, st, g.reshape(1, din), be.reshape(1, din), w,
          b.reshape(1, dout))

    y1, st1 = mid(y0, st0, g0, be0, W1, b1, 128)

    st2 = pl.pallas_call(
        _mlp3_body,
        grid=(_NRB,),
        in_specs=[pl.BlockSpec((_RBLK, 128), lambda i: (i, 0)),
                  _full((8, 128)), _full((1, 128)), _full((1, 128)),
                  _full((128, 256)), _full((1, 256))],
        out_specs=_full((8, 256)),
        out_shape=jax.ShapeDtypeStruct((8, 256), jnp.float32),
    )(y1, st1, g1.reshape(1, 128), be1.reshape(1, 128), W2,
      b2.reshape(1, 256))

    out = pl.pallas_call(
        _mlp_tail_body,
        grid=(_NRB,),
        in_specs=[pl.BlockSpec((_RBLK, 128), lambda i: (i, 0)),
                  _full((8, 128)), _full((1, 128)), _full((1, 128)),
                  _full((128, 256)), _full((1, 256)),
                  _full((8, 256)), _full((1, 256)), _full((1, 256))],
        out_specs=pl.BlockSpec((_RBLK // _NSAMPLE, 256), lambda i: (i, 0)),
        out_shape=jax.ShapeDtypeStruct((_RW // _NSAMPLE, 256), jnp.float32),
    )(y1, st1, g1.reshape(1, 128), be1.reshape(1, 128), W2,
      b2.reshape(1, 256), st2, g2.reshape(1, 256), be2.reshape(1, 256))
    return out


def _index_points(points, idx):
    bsz = points.shape[0]
    out_shape = idx.shape[1:]
    idx_flat = idx.reshape(bsz, -1)
    g = jnp.take_along_axis(points, idx_flat[..., None], axis=1)
    return g.reshape((bsz,) + tuple(out_shape) + (points.shape[-1],))


def _ball_query(radius, nsample, xyz, new_xyz):
    bsz, s, _ = new_xyz.shape
    n = xyz.shape[1]
    sqrdists = (jnp.sum(new_xyz ** 2, axis=-1)[:, :, None]
                + jnp.sum(xyz ** 2, axis=-1)[:, None, :]
                - 2.0 * jnp.einsum('bsd,bnd->bsn', new_xyz, xyz))
    group_idx = jnp.broadcast_to(jnp.arange(n, dtype=jnp.int32), (bsz, s, n))
    group_idx = jnp.where(sqrdists > radius ** 2, n, group_idx)
    group_idx = jnp.sort(group_idx, axis=-1)[:, :, :nsample]
    group_first = group_idx[:, :, 0:1]
    group_idx = jnp.where(group_idx == n,
                          jnp.broadcast_to(group_first, group_idx.shape),
                          group_idx)
    return group_idx


def _mlp_apply(x, params):
    shape = x.shape
    xf = x.reshape(-1, shape[-1])
    for (w, b, g, be) in params:
        xf = xf @ w + b
        m = jnp.mean(xf, axis=0)
        v = jnp.var(xf, axis=0)
        xf = g * (xf - m) / jnp.sqrt(v + 1e-5) + be
        xf = jnp.maximum(xf, 0.0)
    return xf.reshape(tuple(shape[:-1]) + (xf.shape[-1],))


def kernel(xyz, points, W0, b0, g0, be0, W1, b1, g1, be1, W2, b2, g2, be2):
    xyzt = jnp.transpose(xyz, (2, 0, 1))
    newx, newy, newz = _run_fps(xyzt)
    new_xyz = jnp.stack([newx, newy, newz], axis=-1)
    q_pad = jnp.concatenate(
        [new_xyz, jnp.zeros((_B, _NPOINT, 5), jnp.float32)], axis=-1)
    p_pad = jnp.concatenate(
        [jnp.transpose(xyzt, (1, 0, 2)),
         jnp.zeros((_B, 5, _N), jnp.float32)], axis=1)
    words = _run_mask(q_pad, p_pad)
    x1, x0 = _run_ball_group(words, xyzt, newx, newy, newz, points)
    out = _run_mlp(x0, x1, W0, b0, g0, be0, W1, b1, g1, be1, W2, b2, g2, be2)
    new_points = out.reshape(_B, _NPOINT, _MLP_DIMS[-1])
    return (new_xyz, new_points)


# speedup vs baseline: 1.1514x; 1.1514x over previous
import functools

import jax
from jax import lax
import jax.numpy as jnp
from jax.experimental import pallas as pl
from jax.experimental.pallas import tpu as pltpu
from jax.experimental.pallas import tpu_sc as plsc

_NPOINT = 1024
_RADIUS = 0.2
_NSAMPLE = 32
_B, _N, _CFEAT = 8, 4096, 64
_MLP_DIMS = [128, 128, 256]
_IN_CH = 3 + _CFEAT


def _fps_body(xyzt_ref, cx_ref, cy_ref, cz_ref, dist_ref):
    x = xyzt_ref[0]
    y = xyzt_ref[1]
    z = xyzt_ref[2]
    iota_n = jax.lax.broadcasted_iota(jnp.int32, (_B, _N), 1)
    iota_s = jax.lax.broadcasted_iota(jnp.int32, (_B, _NPOINT), 1)
    dist_ref[...] = jnp.full((_B, _N), 1e10, jnp.float32)

    def body(i, carry):
        far, cxa, cya, cza = carry
        onehot = (iota_n == far).astype(jnp.float32)
        cx = jnp.sum(x * onehot, axis=1, keepdims=True)
        cy = jnp.sum(y * onehot, axis=1, keepdims=True)
        cz = jnp.sum(z * onehot, axis=1, keepdims=True)
        sel = iota_s == i
        cxa = jnp.where(sel, cx, cxa)
        cya = jnp.where(sel, cy, cya)
        cza = jnp.where(sel, cz, cza)
        dx = x - cx
        dy = y - cy
        dz = z - cz
        d = (dx * dx + dy * dy) + dz * dz
        dmin = jnp.minimum(dist_ref[...], d)
        dist_ref[...] = dmin
        m = jnp.max(dmin, axis=1, keepdims=True)
        far = jnp.min(jnp.where(dmin == m, iota_n, _N), axis=1, keepdims=True)
        return far, cxa, cya, cza

    far0 = jnp.zeros((_B, 1), jnp.int32)
    zeros_s = jnp.zeros((_B, _NPOINT), jnp.float32)
    _, cxa, cya, cza = jax.lax.fori_loop(
        0, _NPOINT, body, (far0, zeros_s, zeros_s, zeros_s))
    cx_ref[...] = cxa
    cy_ref[...] = cya
    cz_ref[...] = cza


def _run_fps(xyzt, interpret=False):
    cx, cy, cz = pl.pallas_call(
        _fps_body,
        out_shape=[jax.ShapeDtypeStruct((_B, _NPOINT), jnp.float32)] * 3,
        scratch_shapes=[pltpu.VMEM((_B, _N), jnp.float32)],
        interpret=interpret,
    )(xyzt)
    return cx, cy, cz


_SBLK = 256


def _mask_body(qp_ref, pp_ref, lo_ref, hi_ref, out_ref):
    qp = qp_ref[0]
    pp = pp_ref[0]
    qx = qp[:, 0:1]
    qy = qp[:, 1:2]
    qz = qp[:, 2:3]
    qn2 = (qx * qx + qy * qy) + qz * qz
    px = pp[0:1, :]
    py = pp[1:2, :]
    pz = pp[2:3, :]
    pn2 = (px * px + py * py) + pz * pz
    dot = jnp.dot(qp, pp, preferred_element_type=jnp.float32)
    sqr = (qn2 + pn2) - 2.0 * dot
    bit = jnp.where(sqr > _R2, 0.0, 1.0).astype(jnp.bfloat16)
    lo = jnp.dot(bit, lo_ref[...], preferred_element_type=jnp.float32)
    hi = jnp.dot(bit, hi_ref[...], preferred_element_type=jnp.float32)
    out_ref[0] = lo.astype(jnp.int32) | (hi.astype(jnp.int32) << 16)


def _pack_bases():
    n = jnp.arange(_N, dtype=jnp.int32)
    t = jnp.arange(_N // 32, dtype=jnp.int32)
    block = (n[:, None] // 32) == t[None, :]
    jmod = (n % 32)[:, None]
    pw = (1 << (jmod % 16)).astype(jnp.float32)
    lo = jnp.where(block & (jmod < 16), pw, 0.0)
    hi = jnp.where(block & (jmod >= 16), pw, 0.0)
    return lo.astype(jnp.bfloat16), hi.astype(jnp.bfloat16)


def _run_mask(q_pad, p_pad):
    lo, hi = _pack_bases()
    return pl.pallas_call(
        _mask_body,
        grid=(_B, _NPOINT // _SBLK),
        in_specs=[
            pl.BlockSpec((1, _SBLK, 8), lambda b, s: (b, s, 0)),
            pl.BlockSpec((1, 8, _N), lambda b, s: (b, 0, 0)),
            pl.BlockSpec((_N, _N // 32), lambda b, s: (0, 0)),
            pl.BlockSpec((_N, _N // 32), lambda b, s: (0, 0)),
        ],
        out_specs=pl.BlockSpec((1, _SBLK, _N // 32), lambda b, s: (b, s, 0)),
        out_shape=jax.ShapeDtypeStruct((_B, _NPOINT, _N // 32), jnp.int32),
    )(q_pad, p_pad, lo, hi)


_NC, _NS, _L = 2, 16, 16
_NW = _NC * _NS
_QPW = _NPOINT * _B // _NW
_RPW = _QPW * _NSAMPLE
_R2 = _RADIUS * _RADIUS
_GCH = 64
_NBUF = 4


def _bq_body(words_hbm, px_hbm, py_hbm, pz_hbm, qx_hbm, qy_hbm, qz_hbm,
             pts_hbm, x1_hbm, x0_hbm,
             words_v, px_v, py_v, pz_v, qx_v, qy_v, qz_v, nbr_v, gidx_v, x0_v,
             bufs_v, gsems, wsems):
    wid = lax.axis_index("s") * _NC + lax.axis_index("c")
    b = wid // 4
    q0 = (wid % 4) * _QPW
    iota = lax.iota(jnp.int32, _L)
    nwords = _N // 32

    pltpu.sync_copy(words_hbm.at[b, pl.ds(q0, _QPW)], words_v)
    pltpu.sync_copy(px_hbm.at[b], px_v)
    pltpu.sync_copy(py_hbm.at[b], py_v)
    pltpu.sync_copy(pz_hbm.at[b], pz_v)
    pltpu.sync_copy(qx_hbm.at[b, pl.ds(q0, _QPW)], qx_v)
    pltpu.sync_copy(qy_hbm.at[b, pl.ds(q0, _QPW)], qy_v)
    pltpu.sync_copy(qz_hbm.at[b, pl.ds(q0, _QPW)], qz_v)

    minint = jnp.int32(-2147483648)
    zeros16 = jnp.zeros((_L,), jnp.int32)
    for qv in range(_QPW // _L):
        qi = qv * _L + iota
        rowbase = qi * _NSAMPLE

        def scan_cond(carry):
            t, w, cnt = carry
            return jnp.any((cnt < _NSAMPLE)
                           & ((w != 0) | (t < nwords - 1)))

        def scan_body(carry):
            t, w, cnt = carry
            adv = (w == 0) & (t < nwords - 1)
            t = t + jnp.where(adv, 1, 0)
            wn = plsc.load_gather(words_v, [qi, t])
            w = jnp.where(adv, wn, w)
            l = w & (-w)
            lf = (l & 0x7FFFFFFF).astype(jnp.float32)
            expo = lax.shift_right_logical(plsc.bitcast(lf, jnp.int32),
                                           23) - 127
            pos = jnp.where(l == minint, 31, expo)
            m = (l != 0) & (cnt < _NSAMPLE)
            plsc.store_scatter(nbr_v, [rowbase + cnt], t * 32 + pos, mask=m)
            w = w ^ l
            cnt = cnt + jnp.where(m, 1, 0)
            return t, w, cnt

        w0 = plsc.load_gather(words_v, [qi, zeros16])
        _, _, cnt = lax.while_loop(scan_cond, scan_body,
                                   (zeros16, w0, zeros16))
        first = plsc.load_gather(nbr_v, [rowbase])
        for k in range(1, _NSAMPLE):
            plsc.store_scatter(nbr_v, [rowbase + k], first,
                               mask=cnt <= k)

    def cbody(i, _):
        ii = i * _L + iota
        loc = plsc.load_gather(nbr_v, [ii])
        plsc.store_scatter(gidx_v, [ii], loc + b * _N)
        s_loc = lax.shift_right_logical(ii, 5)
        pxn = plsc.load_gather(px_v, [loc])
        pyn = plsc.load_gather(py_v, [loc])
        pzn = plsc.load_gather(pz_v, [loc])
        qxn = plsc.load_gather(qx_v, [s_loc])
        qyn = plsc.load_gather(qy_v, [s_loc])
        qzn = plsc.load_gather(qz_v, [s_loc])
        zero = jnp.zeros((_L,), jnp.float32)
        plsc.store_scatter(x0_v, [ii * 4], pxn - qxn)
        plsc.store_scatter(x0_v, [ii * 4 + 1], pyn - qyn)
        plsc.store_scatter(x0_v, [ii * 4 + 2], pzn - qzn)
        plsc.store_scatter(x0_v, [ii * 4 + 3], zero)
        return 0

    lax.fori_loop(0, _RPW // _L, cbody, 0)
    pltpu.sync_copy(x0_v, x0_hbm.at[pl.ds(wid * _RPW * 4, _RPW * 4)])

    nch = _RPW // _GCH
    gcp = [None] * _NBUF
    wcp = [None] * _NBUF
    for c in range(nch + 2):
        if c < nch:
            if c >= _NBUF:
                wcp[c % _NBUF].wait()
            gcp[c % _NBUF] = pltpu.async_copy(
                pts_hbm.at[gidx_v.at[pl.ds(c * _GCH, _GCH)]],
                bufs_v.at[c % _NBUF], gsems[c % _NBUF])
        if c >= 2:
            p = (c - 2) % _NBUF
            gcp[p].wait()
            wcp[p] = pltpu.async_copy(
                bufs_v.at[p],
                x1_hbm.at[pl.ds(wid * _RPW + (c - 2) * _GCH, _GCH)],
                wsems[p])
    for c in range(nch - _NBUF, nch):
        wcp[c % _NBUF].wait()


def _run_ball_group(words, xyzt, newx, newy, newz, points):
    pts_flat = points.reshape(_B * _N, _CFEAT)
    mesh = plsc.VectorSubcoreMesh(core_axis_name="c", subcore_axis_name="s",
                                  num_cores=_NC, num_subcores=_NS)
    x1, x0 = pl.kernel(
        _bq_body,
        compiler_params=pltpu.CompilerParams(needs_layout_passes=False,
                                             use_tc_tiling_on_sc=False),
        out_type=[
            jax.ShapeDtypeStruct((_B * _NPOINT * _NSAMPLE, _CFEAT),
                                 jnp.float32),
            jax.ShapeDtypeStruct((_B * _NPOINT * _NSAMPLE * 4,), jnp.float32),
        ],
        mesh=mesh,
        scratch_types=[
            pltpu.VMEM((_QPW, _N // 32), jnp.int32),
            pltpu.VMEM((_N,), jnp.float32),
            pltpu.VMEM((_N,), jnp.float32),
            pltpu.VMEM((_N,), jnp.float32),
            pltpu.VMEM((_QPW,), jnp.float32),
            pltpu.VMEM((_QPW,), jnp.float32),
            pltpu.VMEM((_QPW,), jnp.float32),
            pltpu.VMEM((_RPW,), jnp.int32),
            pltpu.VMEM((_RPW,), jnp.int32),
            pltpu.VMEM((_RPW * 4,), jnp.float32),
            pltpu.VMEM((_NBUF, _GCH, _CFEAT), jnp.float32),
            [pltpu.SemaphoreType.DMA] * _NBUF,
            [pltpu.SemaphoreType.DMA] * _NBUF,
        ],
    )(words, xyzt[0], xyzt[1], xyzt[2], newx, newy, newz, pts_flat)
    return x1, x0.reshape(_B * _NPOINT * _NSAMPLE, 4)


_RW = _B * _NPOINT * _NSAMPLE
_RBLK = 4096
_NRB = _RW // _RBLK


def _stats_update(st_ref, y):
    s = jnp.sum(y, axis=0, keepdims=True)
    s2 = jnp.sum(y * y, axis=0, keepdims=True)
    st = jnp.concatenate(
        [s, s2, jnp.zeros((6, y.shape[1]), jnp.float32)], axis=0)

    @pl.when(pl.program_id(0) == 0)
    def _():
        st_ref[...] = jnp.zeros_like(st_ref)

    st_ref[...] += st


def _norm_relu(y, st_ref, g_ref, be_ref):
    n = jnp.float32(_RW)
    mean = st_ref[0:1] / n
    var = st_ref[1:2] / n - mean * mean
    scale = g_ref[...] / jnp.sqrt(var + 1e-5)
    shift = be_ref[...] - mean * scale
    return jnp.maximum(y * scale + shift, 0.0)


def _mlp1_body(x0_ref, x1_ref, w0a_ref, w0b_ref, b0_ref, y_ref, st_ref):
    y = jnp.dot(x1_ref[...], w0b_ref[...],
                preferred_element_type=jnp.float32)
    y = y + jnp.dot(x0_ref[...], w0a_ref[...],
                    preferred_element_type=jnp.float32)
    y = y + b0_ref[...]
    y_ref[...] = y
    _stats_update(st_ref, y)


def _mlp_mid_body(y_ref, st_ref, g_ref, be_ref, w_ref, b_ref,
                  out_ref, st2_ref):
    x = _norm_relu(y_ref[...], st_ref, g_ref, be_ref)
    y = jnp.dot(x, w_ref[...], preferred_element_type=jnp.float32)
    y = y + b_ref[...]
    out_ref[...] = y
    _stats_update(st2_ref, y)


def _mlp3_body(y_ref, st_ref, g_ref, be_ref, w_ref, b_ref, st2_ref):
    x = _norm_relu(y_ref[...], st_ref, g_ref, be_ref)
    y = jnp.dot(x, w_ref[...], preferred_element_type=jnp.float32)
    y = y + b_ref[...]
    _stats_update(st2_ref, y)


def _mlp_tail_body(y_ref, st_ref, g_ref, be_ref, w_ref, b_ref,
                   st2_ref, g2_ref, be2_ref, out_ref):
    x = _norm_relu(y_ref[...], st_ref, g_ref, be_ref)
    y = jnp.dot(x, w_ref[...], preferred_element_type=jnp.float32)
    y = y + b_ref[...]
    x2 = _norm_relu(y, st2_ref, g2_ref, be2_ref)
    xg = x2.reshape(_RBLK // _NSAMPLE, _NSAMPLE, x2.shape[-1])
    out_ref[...] = jnp.max(xg, axis=1)


def _full(shape):
    return pl.BlockSpec(shape, lambda i: tuple(0 for _ in shape))


def _run_mlp(x0, x1, W0, b0, g0, be0, W1, b1, g1, be1, W2, b2, g2, be2):
    w0a = jnp.concatenate([W0[:3], jnp.zeros((1, 128), jnp.float32)], axis=0)
    w0b = W0[3:]
    y0, st0 = pl.pallas_call(
        _mlp1_body,
        grid=(_NRB,),
        in_specs=[pl.BlockSpec((_RBLK, 4), lambda i: (i, 0)),
                  pl.BlockSpec((_RBLK, _CFEAT), lambda i: (i, 0)),
                  _full((4, 128)), _full((_CFEAT, 128)), _full((1, 128))],
        out_specs=[pl.BlockSpec((_RBLK, 128), lambda i: (i, 0)),
                   _full((8, 128))],
        out_shape=[jax.ShapeDtypeStruct((_RW, 128), jnp.float32),
                   jax.ShapeDtypeStruct((8, 128), jnp.float32)],
    )(x0, x1, w0a, w0b, b0.reshape(1, 128))

    def mid(y, st, g, be, w, b, dout):
        din = y.shape[-1]
        return pl.pallas_call(
            _mlp_mid_body,
            grid=(_NRB,),
            in_specs=[pl.BlockSpec((_RBLK, din), lambda i: (i, 0)),
                      _full((8, din)), _full((1, din)), _full((1, din)),
                      _full((din, dout)), _full((1, dout))],
            out_specs=[pl.BlockSpec((_RBLK, dout), lambda i: (i, 0)),
                       _full((8, dout))],
            out_shape=[jax.ShapeDtypeStruct((_RW, dout), jnp.float32),
                       jax.ShapeDtypeStruct((8, dout), jnp.float32)],
        )(y, st, g.reshape(1, din), be.reshape(1, din), w,
          b.reshape(1, dout))

    y1, st1 = mid(y0, st0, g0, be0, W1, b1, 128)

    st2 = pl.pallas_call(
        _mlp3_body,
        grid=(_NRB,),
        in_specs=[pl.BlockSpec((_RBLK, 128), lambda i: (i, 0)),
                  _full((8, 128)), _full((1, 128)), _full((1, 128)),
                  _full((128, 256)), _full((1, 256))],
        out_specs=_full((8, 256)),
        out_shape=jax.ShapeDtypeStruct((8, 256), jnp.float32),
    )(y1, st1, g1.reshape(1, 128), be1.reshape(1, 128), W2,
      b2.reshape(1, 256))

    out = pl.pallas_call(
        _mlp_tail_body,
        grid=(_NRB,),
        in_specs=[pl.BlockSpec((_RBLK, 128), lambda i: (i, 0)),
                  _full((8, 128)), _full((1, 128)), _full((1, 128)),
                  _full((128, 256)), _full((1, 256)),
                  _full((8, 256)), _full((1, 256)), _full((1, 256))],
        out_specs=pl.BlockSpec((_RBLK // _NSAMPLE, 256), lambda i: (i, 0)),
        out_shape=jax.ShapeDtypeStruct((_RW // _NSAMPLE, 256), jnp.float32),
    )(y1, st1, g1.reshape(1, 128), be1.reshape(1, 128), W2,
      b2.reshape(1, 256), st2, g2.reshape(1, 256), be2.reshape(1, 256))
    return out


def _index_points(points, idx):
    bsz = points.shape[0]
    out_shape = idx.shape[1:]
    idx_flat = idx.reshape(bsz, -1)
    g = jnp.take_along_axis(points, idx_flat[..., None], axis=1)
    return g.reshape((bsz,) + tuple(out_shape) + (points.shape[-1],))


def _ball_query(radius, nsample, xyz, new_xyz):
    bsz, s, _ = new_xyz.shape
    n = xyz.shape[1]
    sqrdists = (jnp.sum(new_xyz ** 2, axis=-1)[:, :, None]
                + jnp.sum(xyz ** 2, axis=-1)[:, None, :]
                - 2.0 * jnp.einsum('bsd,bnd->bsn', new_xyz, xyz))
    group_idx = jnp.broadcast_to(jnp.arange(n, dtype=jnp.int32), (bsz, s, n))
    group_idx = jnp.where(sqrdists > radius ** 2, n, group_idx)
    group_idx = jnp.sort(group_idx, axis=-1)[:, :, :nsample]
    group_first = group_idx[:, :, 0:1]
    group_idx = jnp.where(group_idx == n,
                          jnp.broadcast_to(group_first, group_idx.shape),
                          group_idx)
    return group_idx


def _mlp_apply(x, params):
    shape = x.shape
    xf = x.reshape(-1, shape[-1])
    for (w, b, g, be) in params:
        xf = xf @ w + b
        m = jnp.mean(xf, axis=0)
        v = jnp.var(xf, axis=0)
        xf = g * (xf - m) / jnp.sqrt(v + 1e-5) + be
        xf = jnp.maximum(xf, 0.0)
    return xf.reshape(tuple(shape[:-1]) + (xf.shape[-1],))


def kernel(xyz, points, W0, b0, g0, be0, W1, b1, g1, be1, W2, b2, g2, be2):
    xyzt = jnp.transpose(xyz, (2, 0, 1))
    newx, newy, newz = _run_fps(xyzt)
    new_xyz = jnp.stack([newx, newy, newz], axis=-1)
    q_pad = jnp.concatenate(
        [new_xyz, jnp.zeros((_B, _NPOINT, 5), jnp.float32)], axis=-1)
    p_pad = jnp.concatenate(
        [jnp.transpose(xyzt, (1, 0, 2)),
         jnp.zeros((_B, 5, _N), jnp.float32)], axis=1)
    words = _run_mask(q_pad, p_pad)
    x1, x0 = _run_ball_group(words, xyzt, newx, newy, newz, points)
    out = _run_mlp(x0, x1, W0, b0, g0, be0, W1, b1, g1, be1, W2, b2, g2, be2)
    new_points = out.reshape(_B, _NPOINT, _MLP_DIMS[-1])
    return (new_xyz, new_points)


# speedup vs baseline: 1.2043x; 1.0460x over previous
import functools

import jax
from jax import lax
import jax.numpy as jnp
from jax.experimental import pallas as pl
from jax.experimental.pallas import tpu as pltpu
from jax.experimental.pallas import tpu_sc as plsc

_NPOINT = 1024
_RADIUS = 0.2
_NSAMPLE = 32
_B, _N, _CFEAT = 8, 4096, 64
_MLP_DIMS = [128, 128, 256]
_IN_CH = 3 + _CFEAT


def _fps_body(xyzt_ref, cx_ref, cy_ref, cz_ref, dist_ref):
    x = xyzt_ref[0]
    y = xyzt_ref[1]
    z = xyzt_ref[2]
    iota_n = jax.lax.broadcasted_iota(jnp.int32, (_B, _N), 1)
    iota_s = jax.lax.broadcasted_iota(jnp.int32, (_B, _NPOINT), 1)
    dist_ref[...] = jnp.full((_B, _N), 1e10, jnp.float32)

    def body(i, carry):
        far, cxa, cya, cza = carry
        onehot = (iota_n == far).astype(jnp.float32)
        cx = jnp.sum(x * onehot, axis=1, keepdims=True)
        cy = jnp.sum(y * onehot, axis=1, keepdims=True)
        cz = jnp.sum(z * onehot, axis=1, keepdims=True)
        sel = iota_s == i
        cxa = jnp.where(sel, cx, cxa)
        cya = jnp.where(sel, cy, cya)
        cza = jnp.where(sel, cz, cza)
        dx = x - cx
        dy = y - cy
        dz = z - cz
        d = (dx * dx + dy * dy) + dz * dz
        dmin = jnp.minimum(dist_ref[...], d)
        dist_ref[...] = dmin
        m = jnp.max(dmin, axis=1, keepdims=True)
        far = jnp.min(jnp.where(dmin == m, iota_n, _N), axis=1, keepdims=True)
        return far, cxa, cya, cza

    far0 = jnp.zeros((_B, 1), jnp.int32)
    zeros_s = jnp.zeros((_B, _NPOINT), jnp.float32)
    _, cxa, cya, cza = jax.lax.fori_loop(
        0, _NPOINT, body, (far0, zeros_s, zeros_s, zeros_s))
    cx_ref[...] = cxa
    cy_ref[...] = cya
    cz_ref[...] = cza


def _run_fps(xyzt, interpret=False):
    cx, cy, cz = pl.pallas_call(
        _fps_body,
        out_shape=[jax.ShapeDtypeStruct((_B, _NPOINT), jnp.float32)] * 3,
        scratch_shapes=[pltpu.VMEM((_B, _N), jnp.float32)],
        interpret=interpret,
    )(xyzt)
    return cx, cy, cz


_SBLK = 256


def _mask_body(qp_ref, pp_ref, lo_ref, hi_ref, out_ref):
    qp = qp_ref[0]
    pp = pp_ref[0]
    qx = qp[:, 0:1]
    qy = qp[:, 1:2]
    qz = qp[:, 2:3]
    qn2 = (qx * qx + qy * qy) + qz * qz
    px = pp[0:1, :]
    py = pp[1:2, :]
    pz = pp[2:3, :]
    pn2 = (px * px + py * py) + pz * pz
    dot = jnp.dot(qp, pp, preferred_element_type=jnp.float32)
    sqr = (qn2 + pn2) - 2.0 * dot
    bit = jnp.where(sqr > _R2, 0.0, 1.0).astype(jnp.bfloat16)
    lo = jnp.dot(bit, lo_ref[...], preferred_element_type=jnp.float32)
    hi = jnp.dot(bit, hi_ref[...], preferred_element_type=jnp.float32)
    out_ref[0] = lo.astype(jnp.int32) | (hi.astype(jnp.int32) << 16)


def _pack_bases():
    n = jnp.arange(_N, dtype=jnp.int32)
    t = jnp.arange(_N // 32, dtype=jnp.int32)
    block = (n[:, None] // 32) == t[None, :]
    jmod = (n % 32)[:, None]
    pw = (1 << (jmod % 16)).astype(jnp.float32)
    lo = jnp.where(block & (jmod < 16), pw, 0.0)
    hi = jnp.where(block & (jmod >= 16), pw, 0.0)
    return lo.astype(jnp.bfloat16), hi.astype(jnp.bfloat16)


def _run_mask(q_pad, p_pad):
    lo, hi = _pack_bases()
    return pl.pallas_call(
        _mask_body,
        grid=(_B, _NPOINT // _SBLK),
        in_specs=[
            pl.BlockSpec((1, _SBLK, 8), lambda b, s: (b, s, 0)),
            pl.BlockSpec((1, 8, _N), lambda b, s: (b, 0, 0)),
            pl.BlockSpec((_N, _N // 32), lambda b, s: (0, 0)),
            pl.BlockSpec((_N, _N // 32), lambda b, s: (0, 0)),
        ],
        out_specs=pl.BlockSpec((1, _SBLK, _N // 32), lambda b, s: (b, s, 0)),
        out_shape=jax.ShapeDtypeStruct((_B, _NPOINT, _N // 32), jnp.int32),
    )(q_pad, p_pad, lo, hi)


_NC, _NS, _L = 2, 16, 16
_NW = _NC * _NS
_QPW = _NPOINT * _B // _NW
_RPW = _QPW * _NSAMPLE
_R2 = _RADIUS * _RADIUS
_GCH = 64
_NBUF = 4


def _bq_body(words_hbm, px_hbm, py_hbm, pz_hbm, qx_hbm, qy_hbm, qz_hbm,
             pts_hbm, x1_hbm, x0_hbm,
             words_v, px_v, py_v, pz_v, qx_v, qy_v, qz_v, nbr_v, gidx_v, x0_v,
             bufs_v, gsems, wsems):
    wid = lax.axis_index("s") * _NC + lax.axis_index("c")
    b = wid // 4
    q0 = (wid % 4) * _QPW
    iota = lax.iota(jnp.int32, _L)
    nwords = _N // 32

    pltpu.sync_copy(words_hbm.at[b, pl.ds(q0, _QPW)], words_v)
    pltpu.sync_copy(px_hbm.at[b], px_v)
    pltpu.sync_copy(py_hbm.at[b], py_v)
    pltpu.sync_copy(pz_hbm.at[b], pz_v)
    pltpu.sync_copy(qx_hbm.at[b, pl.ds(q0, _QPW)], qx_v)
    pltpu.sync_copy(qy_hbm.at[b, pl.ds(q0, _QPW)], qy_v)
    pltpu.sync_copy(qz_hbm.at[b, pl.ds(q0, _QPW)], qz_v)

    minint = jnp.int32(-2147483648)
    zeros16 = jnp.zeros((_L,), jnp.int32)
    for qv in range(_QPW // _L):
        qi = qv * _L + iota
        rowbase = qi * _NSAMPLE

        def scan_cond(carry):
            t, w, cnt = carry
            return jnp.any((cnt < _NSAMPLE)
                           & ((w != 0) | (t < nwords - 1)))

        def scan_body(carry):
            t, w, cnt = carry
            adv = (w == 0) & (t < nwords - 1)
            t = t + jnp.where(adv, 1, 0)
            wn = plsc.load_gather(words_v, [qi, t])
            w = jnp.where(adv, wn, w)
            l = w & (-w)
            lf = (l & 0x7FFFFFFF).astype(jnp.float32)
            expo = lax.shift_right_logical(plsc.bitcast(lf, jnp.int32),
                                           23) - 127
            pos = jnp.where(l == minint, 31, expo)
            m = (l != 0) & (cnt < _NSAMPLE)
            plsc.store_scatter(nbr_v, [rowbase + cnt], t * 32 + pos, mask=m)
            w = w ^ l
            cnt = cnt + jnp.where(m, 1, 0)
            return t, w, cnt

        w0 = plsc.load_gather(words_v, [qi, zeros16])
        _, _, cnt = lax.while_loop(scan_cond, scan_body,
                                   (zeros16, w0, zeros16))
        first = plsc.load_gather(nbr_v, [rowbase])
        for k in range(1, _NSAMPLE):
            plsc.store_scatter(nbr_v, [rowbase + k], first,
                               mask=cnt <= k)

    def cbody(i, _):
        ii = i * _L + iota
        loc = plsc.load_gather(nbr_v, [ii])
        plsc.store_scatter(gidx_v, [ii], loc + b * _N)
        s_loc = lax.shift_right_logical(ii, 5)
        pxn = plsc.load_gather(px_v, [loc])
        pyn = plsc.load_gather(py_v, [loc])
        pzn = plsc.load_gather(pz_v, [loc])
        qxn = plsc.load_gather(qx_v, [s_loc])
        qyn = plsc.load_gather(qy_v, [s_loc])
        qzn = plsc.load_gather(qz_v, [s_loc])
        zero = jnp.zeros((_L,), jnp.float32)
        plsc.store_scatter(x0_v, [ii * 4], pxn - qxn)
        plsc.store_scatter(x0_v, [ii * 4 + 1], pyn - qyn)
        plsc.store_scatter(x0_v, [ii * 4 + 2], pzn - qzn)
        plsc.store_scatter(x0_v, [ii * 4 + 3], zero)
        return 0

    lax.fori_loop(0, _RPW // _L, cbody, 0)
    pltpu.sync_copy(x0_v, x0_hbm.at[pl.ds(wid * _RPW * 4, _RPW * 4)])

    nch = _RPW // _GCH
    gcp = [None] * _NBUF
    wcp = [None] * _NBUF
    for c in range(nch + 2):
        if c < nch:
            if c >= _NBUF:
                wcp[c % _NBUF].wait()
            gcp[c % _NBUF] = pltpu.async_copy(
                pts_hbm.at[gidx_v.at[pl.ds(c * _GCH, _GCH)]],
                bufs_v.at[c % _NBUF], gsems[c % _NBUF])
        if c >= 2:
            p = (c - 2) % _NBUF
            gcp[p].wait()
            wcp[p] = pltpu.async_copy(
                bufs_v.at[p],
                x1_hbm.at[pl.ds(wid * _RPW + (c - 2) * _GCH, _GCH)],
                wsems[p])
    for c in range(nch - _NBUF, nch):
        wcp[c % _NBUF].wait()


def _run_ball_group(words, xyzt, newx, newy, newz, points):
    pts_flat = points.reshape(_B * _N, _CFEAT)
    mesh = plsc.VectorSubcoreMesh(core_axis_name="c", subcore_axis_name="s",
                                  num_cores=_NC, num_subcores=_NS)
    x1, x0 = pl.kernel(
        _bq_body,
        compiler_params=pltpu.CompilerParams(needs_layout_passes=False,
                                             use_tc_tiling_on_sc=False),
        out_type=[
            jax.ShapeDtypeStruct((_B * _NPOINT * _NSAMPLE, _CFEAT),
                                 jnp.float32),
            jax.ShapeDtypeStruct((_B * _NPOINT * _NSAMPLE * 4,), jnp.float32),
        ],
        mesh=mesh,
        scratch_types=[
            pltpu.VMEM((_QPW, _N // 32), jnp.int32),
            pltpu.VMEM((_N,), jnp.float32),
            pltpu.VMEM((_N,), jnp.float32),
            pltpu.VMEM((_N,), jnp.float32),
            pltpu.VMEM((_QPW,), jnp.float32),
            pltpu.VMEM((_QPW,), jnp.float32),
            pltpu.VMEM((_QPW,), jnp.float32),
            pltpu.VMEM((_RPW,), jnp.int32),
            pltpu.VMEM((_RPW,), jnp.int32),
            pltpu.VMEM((_RPW * 4,), jnp.float32),
            pltpu.VMEM((_NBUF, _GCH, _CFEAT), jnp.float32),
            [pltpu.SemaphoreType.DMA] * _NBUF,
            [pltpu.SemaphoreType.DMA] * _NBUF,
        ],
    )(words, xyzt[0], xyzt[1], xyzt[2], newx, newy, newz, pts_flat)
    return x1, x0.reshape(_B * _NPOINT * _NSAMPLE, 4)


_RW = _B * _NPOINT * _NSAMPLE
_RBLK = 4096
_NRB = _RW // _RBLK


def _stats_update(st_ref, y):
    s = jnp.sum(y, axis=0, keepdims=True)
    s2 = jnp.sum(y * y, axis=0, keepdims=True)
    st = jnp.concatenate(
        [s, s2, jnp.zeros((6, y.shape[1]), jnp.float32)], axis=0)

    @pl.when(pl.program_id(0) == 0)
    def _():
        st_ref[...] = jnp.zeros_like(st_ref)

    st_ref[...] += st


def _norm_relu(y, st_ref, g_ref, be_ref):
    n = jnp.float32(_RW)
    mean = st_ref[0:1] / n
    var = st_ref[1:2] / n - mean * mean
    scale = g_ref[...] / jnp.sqrt(var + 1e-5)
    shift = be_ref[...] - mean * scale
    return jnp.maximum(y * scale + shift, 0.0)


def _mlp1_body(x0_ref, x1_ref, w0a_ref, w0b_ref, b0_ref, y_ref, st_ref):
    y = jnp.dot(x1_ref[...], w0b_ref[...],
                preferred_element_type=jnp.float32)
    y = y + jnp.dot(x0_ref[...], w0a_ref[...],
                    preferred_element_type=jnp.float32)
    y = y + b0_ref[...]
    y_ref[...] = y.astype(jnp.bfloat16)
    _stats_update(st_ref, y)


def _mlp_mid_body(y_ref, st_ref, g_ref, be_ref, w_ref, b_ref,
                  out_ref, st2_ref):
    x = _norm_relu(y_ref[...].astype(jnp.float32), st_ref, g_ref, be_ref)
    y = jnp.dot(x, w_ref[...], preferred_element_type=jnp.float32)
    y = y + b_ref[...]
    out_ref[...] = y.astype(jnp.bfloat16)
    _stats_update(st2_ref, y)


def _mlp3_body(y_ref, st_ref, g_ref, be_ref, w_ref, b_ref, st2_ref):
    x = _norm_relu(y_ref[...].astype(jnp.float32), st_ref, g_ref, be_ref)
    y = jnp.dot(x, w_ref[...], preferred_element_type=jnp.float32)
    y = y + b_ref[...]
    _stats_update(st2_ref, y)


def _mlp_tail_body(y_ref, st_ref, g_ref, be_ref, w_ref, b_ref,
                   st2_ref, g2_ref, be2_ref, out_ref):
    x = _norm_relu(y_ref[...].astype(jnp.float32), st_ref, g_ref, be_ref)
    y = jnp.dot(x, w_ref[...], preferred_element_type=jnp.float32)
    y = y + b_ref[...]
    x2 = _norm_relu(y, st2_ref, g2_ref, be2_ref)
    xg = x2.reshape(_RBLK // _NSAMPLE, _NSAMPLE, x2.shape[-1])
    out_ref[...] = jnp.max(xg, axis=1)


def _full(shape):
    return pl.BlockSpec(shape, lambda i: tuple(0 for _ in shape))


def _run_mlp(x0, x1, W0, b0, g0, be0, W1, b1, g1, be1, W2, b2, g2, be2):
    w0a = jnp.concatenate([W0[:3], jnp.zeros((1, 128), jnp.float32)], axis=0)
    w0b = W0[3:]
    y0, st0 = pl.pallas_call(
        _mlp1_body,
        grid=(_NRB,),
        in_specs=[pl.BlockSpec((_RBLK, 4), lambda i: (i, 0)),
                  pl.BlockSpec((_RBLK, _CFEAT), lambda i: (i, 0)),
                  _full((4, 128)), _full((_CFEAT, 128)), _full((1, 128))],
        out_specs=[pl.BlockSpec((_RBLK, 128), lambda i: (i, 0)),
                   _full((8, 128))],
        out_shape=[jax.ShapeDtypeStruct((_RW, 128), jnp.bfloat16),
                   jax.ShapeDtypeStruct((8, 128), jnp.float32)],
    )(x0, x1, w0a, w0b, b0.reshape(1, 128))

    def mid(y, st, g, be, w, b, dout):
        din = y.shape[-1]
        return pl.pallas_call(
            _mlp_mid_body,
            grid=(_NRB,),
            in_specs=[pl.BlockSpec((_RBLK, din), lambda i: (i, 0)),
                      _full((8, din)), _full((1, din)), _full((1, din)),
                      _full((din, dout)), _full((1, dout))],
            out_specs=[pl.BlockSpec((_RBLK, dout), lambda i: (i, 0)),
                       _full((8, dout))],
            out_shape=[jax.ShapeDtypeStruct((_RW, dout), jnp.bfloat16),
                       jax.ShapeDtypeStruct((8, dout), jnp.float32)],
        )(y, st, g.reshape(1, din), be.reshape(1, din), w,
          b.reshape(1, dout))

    y1, st1 = mid(y0, st0, g0, be0, W1, b1, 128)

    st2 = pl.pallas_call(
        _mlp3_body,
        grid=(_NRB,),
        in_specs=[pl.BlockSpec((_RBLK, 128), lambda i: (i, 0)),
                  _full((8, 128)), _full((1, 128)), _full((1, 128)),
                  _full((128, 256)), _full((1, 256))],
        out_specs=_full((8, 256)),
        out_shape=jax.ShapeDtypeStruct((8, 256), jnp.float32),
    )(y1, st1, g1.reshape(1, 128), be1.reshape(1, 128), W2,
      b2.reshape(1, 256))

    out = pl.pallas_call(
        _mlp_tail_body,
        grid=(_NRB,),
        in_specs=[pl.BlockSpec((_RBLK, 128), lambda i: (i, 0)),
                  _full((8, 128)), _full((1, 128)), _full((1, 128)),
                  _full((128, 256)), _full((1, 256)),
                  _full((8, 256)), _full((1, 256)), _full((1, 256))],
        out_specs=pl.BlockSpec((_RBLK // _NSAMPLE, 256), lambda i: (i, 0)),
        out_shape=jax.ShapeDtypeStruct((_RW // _NSAMPLE, 256), jnp.float32),
    )(y1, st1, g1.reshape(1, 128), be1.reshape(1, 128), W2,
      b2.reshape(1, 256), st2, g2.reshape(1, 256), be2.reshape(1, 256))
    return out


def _index_points(points, idx):
    bsz = points.shape[0]
    out_shape = idx.shape[1:]
    idx_flat = idx.reshape(bsz, -1)
    g = jnp.take_along_axis(points, idx_flat[..., None], axis=1)
    return g.reshape((bsz,) + tuple(out_shape) + (points.shape[-1],))


def _ball_query(radius, nsample, xyz, new_xyz):
    bsz, s, _ = new_xyz.shape
    n = xyz.shape[1]
    sqrdists = (jnp.sum(new_xyz ** 2, axis=-1)[:, :, None]
                + jnp.sum(xyz ** 2, axis=-1)[:, None, :]
                - 2.0 * jnp.einsum('bsd,bnd->bsn', new_xyz, xyz))
    group_idx = jnp.broadcast_to(jnp.arange(n, dtype=jnp.int32), (bsz, s, n))
    group_idx = jnp.where(sqrdists > radius ** 2, n, group_idx)
    group_idx = jnp.sort(group_idx, axis=-1)[:, :, :nsample]
    group_first = group_idx[:, :, 0:1]
    group_idx = jnp.where(group_idx == n,
                          jnp.broadcast_to(group_first, group_idx.shape),
                          group_idx)
    return group_idx


def _mlp_apply(x, params):
    shape = x.shape
    xf = x.reshape(-1, shape[-1])
    for (w, b, g, be) in params:
        xf = xf @ w + b
        m = jnp.mean(xf, axis=0)
        v = jnp.var(xf, axis=0)
        xf = g * (xf - m) / jnp.sqrt(v + 1e-5) + be
        xf = jnp.maximum(xf, 0.0)
    return xf.reshape(tuple(shape[:-1]) + (xf.shape[-1],))


def kernel(xyz, points, W0, b0, g0, be0, W1, b1, g1, be1, W2, b2, g2, be2):
    xyzt = jnp.transpose(xyz, (2, 0, 1))
    newx, newy, newz = _run_fps(xyzt)
    new_xyz = jnp.stack([newx, newy, newz], axis=-1)
    q_pad = jnp.concatenate(
        [new_xyz, jnp.zeros((_B, _NPOINT, 5), jnp.float32)], axis=-1)
    p_pad = jnp.concatenate(
        [jnp.transpose(xyzt, (1, 0, 2)),
         jnp.zeros((_B, 5, _N), jnp.float32)], axis=1)
    words = _run_mask(q_pad, p_pad)
    x1, x0 = _run_ball_group(words, xyzt, newx, newy, newz, points)
    out = _run_mlp(x0, x1, W0, b0, g0, be0, W1, b1, g1, be1, W2, b2, g2, be2)
    new_points = out.reshape(_B, _NPOINT, _MLP_DIMS[-1])
    return (new_xyz, new_points)
